# baseline (speedup 1.0000x reference)
import jax
import jax.numpy as jnp
from jax.experimental import pallas as pl
from jax.experimental.pallas import tpu as pltpu


_B, _S, _D = 4, 4096, 1024
_E = 128
_NCHUNK = 4


def _dma_kernel(x_ref, tab_ref, out_ref, sem_x, sem_t):
    cs = _S // _NCHUNK
    copies = []
    for b in range(_B):
        for c in range(_NCHUNK):
            rows = pl.ds(c * cs, cs)
            copies.append(pltpu.make_async_copy(
                x_ref.at[b, rows, :],
                out_ref.at[b, rows, pl.ds(0, _D)],
                sem_x,
            ))
            copies.append(pltpu.make_async_copy(
                tab_ref.at[rows, :],
                out_ref.at[b, rows, pl.ds(_D, _E)],
                sem_t,
            ))
    for cp in copies:
        cp.start()
    for cp in copies:
        cp.wait()


def kernel(x, embed_table):
    b, s, d = x.shape
    e = embed_table.shape[1]
    return pl.pallas_call(
        _dma_kernel,
        in_specs=[
            pl.BlockSpec(memory_space=pl.ANY),
            pl.BlockSpec(memory_space=pl.ANY),
        ],
        out_specs=pl.BlockSpec(memory_space=pl.ANY),
        out_shape=jax.ShapeDtypeStruct((b, s, d + e), x.dtype),
        scratch_shapes=[pltpu.SemaphoreType.DMA, pltpu.SemaphoreType.DMA],
    )(x, embed_table)


# speedup vs baseline: 50.9970x; 50.9970x over previous
import jax
import jax.numpy as jnp
from jax.experimental import pallas as pl
from jax.experimental.pallas import tpu as pltpu


_B, _S, _D = 4, 4096, 1024
_E = 128
_SB = 2048


def _concat_kernel(x_ref, tab_ref, out_ref):
    out_ref[:, :, :_D] = x_ref[...]
    out_ref[:, :, _D:] = tab_ref[...][None, :, :]


def kernel(x, embed_table):
    b, s, d = x.shape
    e = embed_table.shape[1]
    grid = (s // _SB, b)
    return pl.pallas_call(
        _concat_kernel,
        grid=grid,
        in_specs=[
            pl.BlockSpec((1, _SB, d), lambda i, j: (j, i, 0)),
            pl.BlockSpec((_SB, e), lambda i, j: (i, 0)),
        ],
        out_specs=pl.BlockSpec((1, _SB, d + e), lambda i, j: (j, i, 0)),
        out_shape=jax.ShapeDtypeStruct((b, s, d + e), x.dtype),
        compiler_params=pltpu.CompilerParams(
            dimension_semantics=("parallel", "parallel"),
        ),
    )(x, embed_table)
